# baseline (speedup 1.0000x reference)
import jax
import jax.numpy as jnp
from jax import lax
from jax.experimental import pallas as pl
from jax.experimental.pallas import tpu as pltpu
from jax.experimental.pallas import tpu_sc as plsc

N = 10000
E = 320000
D = 128
D_EDGE = 16

NC = 2
NS = 16
NW = NC * NS
EPT = E // NW
CHUNK = 128
NBT = E // CHUNK
NJ = 79
NJ_LAST = NBT - (NW - 1) * NJ
STRIPE = 640
NPAD = NS * STRIPE


def _matmul_body(x_ref, fet_ref, ws_ref, wd_ref, b_ref, was_ref, waet_ref,
                 fcs_ref, fcd_ref, asrc_ref, aet_ref):
    x = x_ref[...]
    fcs_ref[...] = jnp.dot(x, ws_ref[...], preferred_element_type=jnp.float32)
    fcd_ref[...] = jnp.dot(x, wd_ref[...], preferred_element_type=jnp.float32) + b_ref[...]
    asrc_ref[...] = jnp.dot(x, was_ref[...], preferred_element_type=jnp.float32)
    ae = jnp.dot(waet_ref[...], fet_ref[...], preferred_element_type=jnp.float32)
    aet_ref[...] = ae[0:1, :]


def _sc_body(asrc_hbm, ei_hbm, ae_hbm, fc_hbm,
             part_hbm, s_hbm,
             eb, aeb, avidx, avb, exb, rows_v, zero1_v,
             acc_sh, s_sh, sem, semw, semi):
    cid = lax.axis_index("c")
    sid = lax.axis_index("s")
    wid = cid * NS + sid
    ebase = wid * NJ
    njw = jnp.where(wid == NW - 1, NJ_LAST, NJ)
    base = pl.multiple_of(sid * STRIPE, STRIPE)

    z16 = jnp.zeros((16,), jnp.float32)

    def zrows(i, c):
        rows_v[0, i // 8, pl.ds((i % 8) * 16, 16)] = z16
        return c
    lax.fori_loop(0, CHUNK * 8, zrows, 0)

    def z1(i, c):
        zero1_v[pl.ds(i * 16, 16)] = z16
        return c
    lax.fori_loop(0, STRIPE // 16, z1, 0)

    for b in range(STRIPE // CHUNK):
        pltpu.sync_copy(rows_v.at[0], acc_sh.at[pl.ds(base + b * CHUNK, CHUNK)])
    pltpu.sync_copy(zero1_v, s_sh.at[pl.ds(base, STRIPE)])

    plsc.subcore_barrier()

    def _stage(m, g):
        off = g * CHUNK
        pltpu.async_copy(ei_hbm.at[0].at[pl.ds(off, CHUNK)], eb.at[2 * m], semi)
        pltpu.async_copy(ei_hbm.at[1].at[pl.ds(off, CHUNK)], eb.at[2 * m + 1], semi)
        pltpu.async_copy(ae_hbm.at[0].at[pl.ds(off, CHUNK)], aeb.at[m], semi)

    def _wait_stage(m, g):
        off = g * CHUNK
        pltpu.make_async_copy(
            ei_hbm.at[0].at[pl.ds(off, CHUNK)], eb.at[2 * m], semi).wait()
        pltpu.make_async_copy(
            ei_hbm.at[1].at[pl.ds(off, CHUNK)], eb.at[2 * m + 1], semi).wait()
        pltpu.make_async_copy(
            ae_hbm.at[0].at[pl.ds(off, CHUNK)], aeb.at[m], semi).wait()

    _stage(0, ebase)
    _wait_stage(0, ebase)
    for k in range(8):
        o = k * 16
        avidx[0, pl.ds(o, 16)] = eb[0, pl.ds(o, 16)] * 8
    pltpu.async_copy(asrc_hbm.at[avidx.at[0]], avb.at[0], sem)
    pltpu.async_copy(fc_hbm.at[eb.at[0]], rows_v.at[0], sem)
    _stage(1, ebase + 1)

    def _ex_scale(b, m):
        for k in range(8):
            o = k * 16
            exb[b, pl.ds(o, 16)] = jnp.exp(avb[b, pl.ds(o, 16)] + aeb[m, pl.ds(o, 16)])

        def scale_body(r8, c2):
            for u in range(8):
                r = r8 * 8 + u
                a = plsc.load_gather(exb.at[b], [jnp.full((16,), r, jnp.int32)])
                for k in range(8):
                    rows_v[b, r, pl.ds(k * 16, 16)] = (
                        rows_v[b, r, pl.ds(k * 16, 16)] * a)
            return c2
        lax.fori_loop(0, CHUNK // 8, scale_body, 0)

    def _wait_gathers(b, m):
        pltpu.make_async_copy(asrc_hbm.at[avidx.at[b]], avb.at[b], sem).wait()
        pltpu.make_async_copy(fc_hbm.at[eb.at[2 * m]], rows_v.at[b], sem).wait()

    def _drain_scatters(b, m):
        pltpu.make_async_copy(exb.at[b], s_sh.at[eb.at[2 * m + 1]], semw).wait()
        pltpu.make_async_copy(
            rows_v.at[b], acc_sh.at[eb.at[2 * m + 1]], semw).wait()

    def _launch_gathers(b, m):
        for k in range(8):
            o = k * 16
            avidx[b, pl.ds(o, 16)] = eb[2 * m, pl.ds(o, 16)] * 8
        pltpu.async_copy(asrc_hbm.at[avidx.at[b]], avb.at[b], sem)
        pltpu.async_copy(fc_hbm.at[eb.at[2 * m]], rows_v.at[b], sem)

    def _issue_scatters(b, m):
        pltpu.async_copy(exb.at[b], s_sh.at[eb.at[2 * m + 1]], semw, add=True)
        pltpu.async_copy(
            rows_v.at[b], acc_sh.at[eb.at[2 * m + 1]], semw, add=True)

    def pair_body(t, c):
        p2 = lax.rem(t, 2) * 2
        m0 = p2
        m1 = p2 + 1
        q = 2 - p2
        pm = 3 - p2

        _wait_gathers(0, m0)

        @pl.when(t >= 1)
        def _():
            _drain_scatters(1, pm)
        _wait_stage(m1, ebase + 2 * t + 1)
        _launch_gathers(1, m1)
        _stage(q, ebase + 2 * t + 2)
        _ex_scale(0, m0)
        _issue_scatters(0, m0)

        _wait_gathers(1, m1)
        _drain_scatters(0, m0)
        _wait_stage(q, ebase + 2 * t + 2)
        _launch_gathers(0, q)

        @pl.when(2 * t + 3 < njw)
        def _():
            _stage(pm, ebase + 2 * t + 3)
        _ex_scale(1, m1)
        _issue_scatters(1, m1)
        return c
    lax.fori_loop(0, (njw - 1) // 2, pair_body, 0)

    _wait_gathers(0, 2)
    _drain_scatters(1, 1)
    _ex_scale(0, 2)
    _issue_scatters(0, 2)
    _drain_scatters(0, 2)

    plsc.subcore_barrier()

    pltpu.sync_copy(acc_sh.at[pl.ds(base, STRIPE)],
                    part_hbm.at[cid].at[pl.ds(base, STRIPE)])
    pltpu.sync_copy(s_sh.at[pl.ds(base, STRIPE)],
                    s_hbm.at[cid].at[pl.ds(base, STRIPE)])


def _epilogue_body(p_ref, s0_ref, s1_ref, fcd_ref, out_ref):
    p = p_ref[...]
    s = s0_ref[...] + s1_ref[...]
    r = 1.0 / (s + 1e-16)
    out_ref[...] = (p[0] + p[1]) * r + fcd_ref[...]


@jax.jit
def kernel(feat_src, edge_index, feat_edge, W_src, W_dst, b_dst, W_attn_src, W_attn_edge):
    was_p = jnp.pad(W_attn_src, ((0, 0), (0, 7)))
    waet_p = jnp.pad(W_attn_edge.T, ((0, 7), (0, 0)))
    b2 = b_dst.reshape(1, D)
    fe_t = feat_edge.T

    g = 25
    bn = N // g
    be = E // g
    fc_src, fc_dst, asrc8, aet = pl.pallas_call(
        _matmul_body,
        grid=(g,),
        in_specs=[
            pl.BlockSpec((bn, D), lambda i: (i, 0)),
            pl.BlockSpec((D_EDGE, be), lambda i: (0, i)),
            pl.BlockSpec((D, D), lambda i: (0, 0)),
            pl.BlockSpec((D, D), lambda i: (0, 0)),
            pl.BlockSpec((1, D), lambda i: (0, 0)),
            pl.BlockSpec((D, 8), lambda i: (0, 0)),
            pl.BlockSpec((8, D_EDGE), lambda i: (0, 0)),
        ],
        out_specs=[
            pl.BlockSpec((bn, D), lambda i: (i, 0)),
            pl.BlockSpec((bn, D), lambda i: (i, 0)),
            pl.BlockSpec((bn, 8), lambda i: (i, 0)),
            pl.BlockSpec((1, be), lambda i: (0, i)),
        ],
        out_shape=[
            jax.ShapeDtypeStruct((N, D), jnp.float32),
            jax.ShapeDtypeStruct((N, D), jnp.float32),
            jax.ShapeDtypeStruct((N, 8), jnp.float32),
            jax.ShapeDtypeStruct((1, E), jnp.float32),
        ],
    )(feat_src, fe_t, W_src, W_dst, b2, was_p, waet_p)

    asrc = asrc8.reshape(N * 8)
    aedge = aet[0]


    sc_fn = pl.kernel(
        _sc_body,
        out_type=(
            jax.ShapeDtypeStruct((NC, NPAD, D), jnp.float32),
            jax.ShapeDtypeStruct((NC, NPAD), jnp.float32),
        ),
        mesh=plsc.VectorSubcoreMesh(core_axis_name="c", subcore_axis_name="s"),
        compiler_params=pltpu.CompilerParams(needs_layout_passes=False),
        scratch_types=[
            pltpu.VMEM((8, CHUNK), jnp.int32),
            pltpu.VMEM((4, CHUNK), jnp.float32),
            pltpu.VMEM((2, CHUNK), jnp.int32),
            pltpu.VMEM((2, CHUNK), jnp.float32),
            pltpu.VMEM((2, CHUNK), jnp.float32),
            pltpu.VMEM((2, CHUNK, D), jnp.float32),
            pltpu.VMEM((STRIPE,), jnp.float32),
            pltpu.VMEM_SHARED((NPAD, D), jnp.float32),
            pltpu.VMEM_SHARED((NPAD,), jnp.float32),
            pltpu.SemaphoreType.DMA,
            pltpu.SemaphoreType.DMA,
            pltpu.SemaphoreType.DMA,
        ],
    )
    part, s_part = sc_fn(asrc, edge_index, aet, fc_src)

    ge = 10
    bo = N // ge
    out = pl.pallas_call(
        _epilogue_body,
        grid=(ge,),
        in_specs=[
            pl.BlockSpec((NC, bo, D), lambda i: (0, i, 0)),
            pl.BlockSpec((bo, 1), lambda i: (i, 0)),
            pl.BlockSpec((bo, 1), lambda i: (i, 0)),
            pl.BlockSpec((bo, D), lambda i: (i, 0)),
        ],
        out_specs=pl.BlockSpec((bo, D), lambda i: (i, 0)),
        out_shape=jax.ShapeDtypeStruct((N, D), jnp.float32),
    )(part,
      s_part[0, :N].reshape(N, 1), s_part[1, :N].reshape(N, 1), fc_dst)

    return out.reshape(N, 1, D)
